# Initial kernel scaffold; baseline (speedup 1.0000x reference)
import jax
import jax.numpy as jnp
from jax.experimental import pallas as pl


def kernel(t, y, x, query_y, query_x, query_t, T, X, Y):
    raise NotImplementedError("write your pallas kernel here")


# speedup vs baseline: 12.6958x; 12.6958x over previous
import functools
import jax
import jax.numpy as jnp
from jax import lax
from jax.experimental import pallas as pl
from jax.experimental.pallas import tpu as pltpu

HEIGHT = 480
WIDTH = 640
D = 64
TAU = 0.1
K = 17
R = 8
PW = WIDTH + 2 * R
GROWS = (HEIGHT + 1) * PW
C2 = 2 * D
CC = C2 + 16

EB = 2048
QB = 2048
RB = 8


def _embed_body(t_ref, T_ref, out_ref):
    tv = t_ref[0, 0, :] * (1.0 / TAU)
    ang = tv[:, None] * T_ref[0, :][None, :]
    out_ref[...] = jnp.concatenate([jnp.cos(ang), jnp.sin(ang)], axis=1)


def _embed(t_pad, T_row, n_ev):
    nb = n_ev // EB
    t3 = t_pad.reshape(nb, 1, EB)
    return pl.pallas_call(
        _embed_body,
        grid=(nb,),
        in_specs=[
            pl.BlockSpec((1, 1, EB), lambda i: (i, 0, 0)),
            pl.BlockSpec((1, D), lambda i: (0, 0)),
        ],
        out_specs=pl.BlockSpec((EB, C2), lambda i: (i, 0)),
        out_shape=jax.ShapeDtypeStruct((n_ev, C2), jnp.float32),
    )(t3, T_row)


def _hconv_body(g_ref, c_ref, w1_ref, w2_ref, h_ref, hc_ref):
    i = pl.program_id(0)
    is_edge = jnp.logical_or(i == 0, i == 61)

    @pl.when(is_edge)
    def _zero():
        h_ref[...] = jnp.zeros_like(h_ref)
        hc_ref[...] = jnp.zeros_like(hc_ref)

    @pl.when(jnp.logical_not(is_edge))
    def _conv():
        g = g_ref[...]
        gs = jnp.concatenate([g[:, :, D:], g[:, :, :D]], axis=2)
        cnt = c_ref[...]
        acc = jnp.zeros((RB, WIDTH, C2), jnp.float32)
        cacc = jnp.zeros((RB, WIDTH, 1), jnp.float32)
        for dx in range(K):
            w1 = w1_ref[dx, :][None, None, :]
            w2 = w2_ref[dx, :][None, None, :]
            acc = acc + g[:, dx:dx + WIDTH, :] * w1
            acc = acc + gs[:, dx:dx + WIDTH, :] * w2
            cacc = cacc + cnt[:, dx:dx + WIDTH, :]
        h_ref[...] = acc
        hc_ref[...] = cacc


def _hconv(grid3, cnt3, w1x, w2x):
    nb = HEIGHT // RB
    return pl.pallas_call(
        _hconv_body,
        grid=(nb + 2,),
        in_specs=[
            pl.BlockSpec((RB, PW, C2),
                         lambda i: (jnp.clip(i - 1, 0, 59), 0, 0)),
            pl.BlockSpec((RB, PW, 1), lambda i: (jnp.clip(i - 1, 0, 59), 0, 0)),
            pl.BlockSpec((K, C2), lambda i: (0, 0)),
            pl.BlockSpec((K, C2), lambda i: (0, 0)),
        ],
        out_specs=[
            pl.BlockSpec((RB, WIDTH, C2), lambda i: (i, 0, 0)),
            pl.BlockSpec((RB, WIDTH, 1), lambda i: (i, 0, 0)),
        ],
        out_shape=[
            jax.ShapeDtypeStruct((HEIGHT + 2 * R, WIDTH, C2), jnp.float32),
            jax.ShapeDtypeStruct((HEIGHT + 2 * R, WIDTH, 1), jnp.float32),
        ],
    )(grid3, cnt3, w1x, w2x)


def _vconv_body(a_ref, b_ref, c_ref, ca_ref, cb_ref, cc_ref,
                w1_ref, w2_ref, out_ref, cout_ref):
    st = jnp.concatenate([a_ref[...], b_ref[...], c_ref[...]], axis=0)
    ss = jnp.concatenate([st[:, :, D:], st[:, :, :D]], axis=2)
    cst = jnp.concatenate([ca_ref[...], cb_ref[...], cc_ref[...]], axis=0)
    acc = jnp.zeros((RB, WIDTH, C2), jnp.float32)
    cacc = jnp.zeros((RB, WIDTH, 1), jnp.float32)
    for oy in range(K):
        w1 = w1_ref[oy, :][None, None, :]
        w2 = w2_ref[oy, :][None, None, :]
        acc = acc + st[oy:oy + RB] * w1
        acc = acc + ss[oy:oy + RB] * w2
        cacc = cacc + cst[oy:oy + RB]
    out_ref[...] = acc
    cout_ref[...] = cacc


def _vconv(H, Hc, w1y, w2y):
    nb = HEIGHT // RB
    hspec = [pl.BlockSpec((RB, WIDTH, C2), (lambda o: (lambda i: (i + o, 0, 0)))(o))
             for o in range(3)]
    cspec = [pl.BlockSpec((RB, WIDTH, 1), (lambda o: (lambda i: (i + o, 0, 0)))(o))
             for o in range(3)]
    return pl.pallas_call(
        _vconv_body,
        grid=(nb,),
        in_specs=hspec + cspec + [
            pl.BlockSpec((K, C2), lambda i: (0, 0)),
            pl.BlockSpec((K, C2), lambda i: (0, 0)),
        ],
        out_specs=[
            pl.BlockSpec((RB, WIDTH, C2), lambda i: (i, 0, 0)),
            pl.BlockSpec((RB, WIDTH, 1), lambda i: (i, 0, 0)),
        ],
        out_shape=[
            jax.ShapeDtypeStruct((HEIGHT, WIDTH, C2), jnp.float32),
            jax.ShapeDtypeStruct((HEIGHT, WIDTH, 1), jnp.float32),
        ],
    )(H, H, H, Hc, Hc, Hc, w1y, w2y)


def _recenter_body(qt_ref, g_ref, T_ref, out_ref):
    qv = qt_ref[0, 0, :] * (1.0 / TAU)
    ang = qv[:, None] * T_ref[0, :][None, :]
    c = jnp.cos(ang)
    s = jnp.sin(ang)
    g = g_ref[...]
    gre = g[:, :D]
    gim = g[:, D:C2]
    cnt = g[:, C2]
    f = (D ** 0.5) / jnp.maximum(cnt, 1.0)
    fre = (gre * c + gim * s) * f[:, None]
    fim = (gim * c - gre * s) * f[:, None]
    out_ref[...] = jnp.concatenate([fre, fim], axis=1)


def _recenter(qt_pad, G, T_row, nq):
    nb = nq // QB
    qt3 = qt_pad.reshape(nb, 1, QB)
    return pl.pallas_call(
        _recenter_body,
        grid=(nb,),
        in_specs=[
            pl.BlockSpec((1, 1, QB), lambda i: (i, 0, 0)),
            pl.BlockSpec((QB, CC), lambda i: (i, 0)),
            pl.BlockSpec((1, D), lambda i: (0, 0)),
        ],
        out_specs=pl.BlockSpec((QB, C2), lambda i: (i, 0)),
        out_shape=jax.ShapeDtypeStruct((nq, C2), jnp.float32),
    )(qt3, G, T_row)


def kernel(t, y, x, query_y, query_x, query_t, T, X, Y):
    t = t.astype(jnp.float32)
    T_row = T.reshape(1, D)
    r = jnp.arange(-R, R + 1, dtype=jnp.float32) / R
    ax = jnp.cos(r[:, None] * X)
    bx = jnp.sin(r[:, None] * X)
    ay = jnp.cos(r[:, None] * Y)
    by = jnp.sin(r[:, None] * Y)
    w1x = jnp.concatenate([ax, ax], axis=1)
    w2x = jnp.concatenate([-bx, bx], axis=1)
    w1y = jnp.concatenate([ay, ay], axis=1)
    w2y = jnp.concatenate([-by, by], axis=1)

    n_ev = t.shape[0]
    n_ev_pad = ((n_ev + EB - 1) // EB) * EB
    pad_e = n_ev_pad - n_ev
    t_pad = jnp.concatenate([t, jnp.zeros((pad_e,), jnp.float32)])
    flat_p = (y * PW + x + R).astype(jnp.int32)
    dump = HEIGHT * PW + (jnp.arange(pad_e, dtype=jnp.int32) % 64)
    idx_pad = jnp.concatenate([flat_p, dump])
    ones_pad = jnp.concatenate(
        [jnp.ones((n_ev,), jnp.float32), jnp.zeros((pad_e,), jnp.float32)])

    emb = _embed(t_pad, T_row, n_ev_pad)

    grid_flat = jnp.zeros((GROWS, C2), jnp.float32).at[idx_pad].add(emb)
    cnt_flat = jnp.zeros((GROWS,), jnp.float32).at[idx_pad].add(ones_pad)

    grid3 = grid_flat.reshape(HEIGHT + 1, PW, C2)
    cnt3 = cnt_flat.reshape(HEIGHT + 1, PW, 1)

    H, Hc = _hconv(grid3, cnt3, w1x, w2x)
    CV, Cc = _vconv(H, Hc, w1y, w2y)
    CV2 = CV.reshape(HEIGHT * WIDTH, C2)
    Cc2 = Cc.reshape(HEIGHT * WIDTH, 1)

    nq = query_y.shape[0]
    nq_pad = ((nq + QB - 1) // QB) * QB
    pad_q = nq_pad - nq
    qidx = (query_y * WIDTH + query_x).astype(jnp.int32)
    qidx_pad = jnp.concatenate(
        [qidx, jnp.arange(pad_q, dtype=jnp.int32)])
    qt_pad = jnp.concatenate(
        [query_t.astype(jnp.float32), jnp.zeros((pad_q,), jnp.float32)])

    G = jnp.pad(jnp.concatenate([CV2[qidx_pad], Cc2[qidx_pad]], axis=1),
                ((0, 0), (0, CC - C2 - 1)))

    O = _recenter(qt_pad, G, T_row, nq_pad)
    out = lax.complex(O[:nq, :D], O[:nq, D:])
    return out


# speedup vs baseline: 12.7367x; 1.0032x over previous
import functools
import jax
import jax.numpy as jnp
from jax import lax
from jax.experimental import pallas as pl
from jax.experimental.pallas import tpu as pltpu

HEIGHT = 480
WIDTH = 640
D = 64
TAU = 0.1
K = 17
R = 8
PW = WIDTH + 2 * R
GY = 496
GROWS = GY * PW
HALF = GROWS // 2
STRIPE = HALF // 16
C2 = 2 * D
CC = C2 + 16
SCH = 8
NCH = C2 // SCH
EW = 7936
SUBW = EW // 128
EALIGN = 16384

EB = 2048
QB = 2048
RB = 8


def _embed_body(t_ref, T_ref, out_ref):
    tv = t_ref[0, 0, :] * (1.0 / TAU)
    ang = tv[:, None] * T_ref[0, :][None, :]
    out_ref[...] = jnp.concatenate([jnp.cos(ang), jnp.sin(ang)], axis=1)


def _embed(t_pad, T_row, n_ev):
    nb = n_ev // EB
    t3 = t_pad.reshape(nb, 1, EB)
    return pl.pallas_call(
        _embed_body,
        grid=(nb,),
        in_specs=[
            pl.BlockSpec((1, 1, EB), lambda i: (i, 0, 0)),
            pl.BlockSpec((1, D), lambda i: (0, 0)),
        ],
        out_specs=pl.BlockSpec((EB, C2), lambda i: (i, 0)),
        out_shape=jax.ShapeDtypeStruct((n_ev, C2), jnp.float32),
    )(t3, T_row)


def _hconv_body(g_ref, c_ref, w1_ref, w2_ref, h_ref, hc_ref):
    i = pl.program_id(0)
    is_edge = jnp.logical_or(i == 0, i == 61)

    @pl.when(is_edge)
    def _zero():
        h_ref[...] = jnp.zeros_like(h_ref)
        hc_ref[...] = jnp.zeros_like(hc_ref)

    @pl.when(jnp.logical_not(is_edge))
    def _conv():
        g = g_ref[...]
        gs = jnp.concatenate([g[:, :, D:], g[:, :, :D]], axis=2)
        cnt = c_ref[...]
        acc = jnp.zeros((RB, WIDTH, C2), jnp.float32)
        cacc = jnp.zeros((RB, WIDTH, 1), jnp.float32)
        for dx in range(K):
            w1 = w1_ref[dx, :][None, None, :]
            w2 = w2_ref[dx, :][None, None, :]
            acc = acc + g[:, dx:dx + WIDTH, :] * w1
            acc = acc + gs[:, dx:dx + WIDTH, :] * w2
            cacc = cacc + cnt[:, dx:dx + WIDTH, :]
        h_ref[...] = acc
        hc_ref[...] = cacc


def _hconv(grid3, cnt3, w1x, w2x):
    nb = HEIGHT // RB
    return pl.pallas_call(
        _hconv_body,
        grid=(nb + 2,),
        in_specs=[
            pl.BlockSpec((RB, PW, C2),
                         lambda i: (jnp.clip(i - 1, 0, 59), 0, 0)),
            pl.BlockSpec((RB, PW, 1), lambda i: (jnp.clip(i - 1, 0, 59), 0, 0)),
            pl.BlockSpec((K, C2), lambda i: (0, 0)),
            pl.BlockSpec((K, C2), lambda i: (0, 0)),
        ],
        out_specs=[
            pl.BlockSpec((RB, WIDTH, C2), lambda i: (i, 0, 0)),
            pl.BlockSpec((RB, WIDTH, 1), lambda i: (i, 0, 0)),
        ],
        out_shape=[
            jax.ShapeDtypeStruct((HEIGHT + 2 * R, WIDTH, C2), jnp.float32),
            jax.ShapeDtypeStruct((HEIGHT + 2 * R, WIDTH, 1), jnp.float32),
        ],
    )(grid3, cnt3, w1x, w2x)


def _vconv_body(a_ref, b_ref, c_ref, ca_ref, cb_ref, cc_ref,
                w1_ref, w2_ref, out_ref, cout_ref):
    st = jnp.concatenate([a_ref[...], b_ref[...], c_ref[...]], axis=0)
    ss = jnp.concatenate([st[:, :, D:], st[:, :, :D]], axis=2)
    cst = jnp.concatenate([ca_ref[...], cb_ref[...], cc_ref[...]], axis=0)
    acc = jnp.zeros((RB, WIDTH, C2), jnp.float32)
    cacc = jnp.zeros((RB, WIDTH, 1), jnp.float32)
    for oy in range(K):
        w1 = w1_ref[oy, :][None, None, :]
        w2 = w2_ref[oy, :][None, None, :]
        acc = acc + st[oy:oy + RB] * w1
        acc = acc + ss[oy:oy + RB] * w2
        cacc = cacc + cst[oy:oy + RB]
    out_ref[...] = acc
    cout_ref[...] = cacc


def _vconv(H, Hc, w1y, w2y):
    nb = HEIGHT // RB
    hspec = [pl.BlockSpec((RB, WIDTH, C2), (lambda o: (lambda i: (i + o, 0, 0)))(o))
             for o in range(3)]
    cspec = [pl.BlockSpec((RB, WIDTH, 1), (lambda o: (lambda i: (i + o, 0, 0)))(o))
             for o in range(3)]
    return pl.pallas_call(
        _vconv_body,
        grid=(nb,),
        in_specs=hspec + cspec + [
            pl.BlockSpec((K, C2), lambda i: (0, 0)),
            pl.BlockSpec((K, C2), lambda i: (0, 0)),
        ],
        out_specs=[
            pl.BlockSpec((RB, WIDTH, C2), lambda i: (i, 0, 0)),
            pl.BlockSpec((RB, WIDTH, 1), lambda i: (i, 0, 0)),
        ],
        out_shape=[
            jax.ShapeDtypeStruct((HEIGHT, WIDTH, C2), jnp.float32),
            jax.ShapeDtypeStruct((HEIGHT, WIDTH, 1), jnp.float32),
        ],
    )(H, H, H, Hc, Hc, Hc, w1y, w2y)


def _recenter_body(qt_ref, g_ref, T_ref, out_ref):
    qv = qt_ref[0, 0, :] * (1.0 / TAU)
    ang = qv[:, None] * T_ref[0, :][None, :]
    c = jnp.cos(ang)
    s = jnp.sin(ang)
    g = g_ref[...]
    gre = g[:, :D]
    gim = g[:, D:C2]
    cnt = g[:, C2]
    f = (D ** 0.5) / jnp.maximum(cnt, 1.0)
    fre = (gre * c + gim * s) * f[:, None]
    fim = (gim * c - gre * s) * f[:, None]
    out_ref[...] = jnp.concatenate([fre, fim], axis=1)


def _recenter(qt_pad, G, T_row, nq):
    nb = nq // QB
    qt3 = qt_pad.reshape(nb, 1, QB)
    return pl.pallas_call(
        _recenter_body,
        grid=(nb,),
        in_specs=[
            pl.BlockSpec((1, 1, QB), lambda i: (i, 0, 0)),
            pl.BlockSpec((QB, CC), lambda i: (i, 0)),
            pl.BlockSpec((1, D), lambda i: (0, 0)),
        ],
        out_specs=pl.BlockSpec((QB, C2), lambda i: (i, 0)),
        out_shape=jax.ShapeDtypeStruct((nq, C2), jnp.float32),
    )(qt3, G, T_row)


def kernel(t, y, x, query_y, query_x, query_t, T, X, Y):
    t = t.astype(jnp.float32)
    T_row = T.reshape(1, D)
    r = jnp.arange(-R, R + 1, dtype=jnp.float32) / R
    ax = jnp.cos(r[:, None] * X)
    bx = jnp.sin(r[:, None] * X)
    ay = jnp.cos(r[:, None] * Y)
    by = jnp.sin(r[:, None] * Y)
    w1x = jnp.concatenate([ax, ax], axis=1)
    w2x = jnp.concatenate([-bx, bx], axis=1)
    w1y = jnp.concatenate([ay, ay], axis=1)
    w2y = jnp.concatenate([-by, by], axis=1)

    n_ev = t.shape[0]
    n_ev_pad = ((n_ev + EB - 1) // EB) * EB
    pad_e = n_ev_pad - n_ev
    t_pad = jnp.concatenate([t, jnp.zeros((pad_e,), jnp.float32)])
    flat_p = (y * PW + x + R).astype(jnp.int32)
    dump = jnp.full((pad_e,), GROWS, jnp.int32)
    idx_pad = jnp.concatenate([flat_p, dump])

    emb = _embed(t_pad, T_row, n_ev_pad)

    grid_flat = jnp.zeros((GROWS, C2), jnp.float32).at[idx_pad].add(emb)
    cnt_flat = jnp.zeros((GROWS,), jnp.float32).at[idx_pad].add(1.0)

    grid3 = grid_flat.reshape(GY, PW, C2)
    cnt3 = cnt_flat.reshape(GY, PW, 1)

    H, Hc = _hconv(grid3, cnt3, w1x, w2x)
    CV, Cc = _vconv(H, Hc, w1y, w2y)
    CV2 = CV.reshape(HEIGHT * WIDTH, C2)
    Cc2 = Cc.reshape(HEIGHT * WIDTH, 1)

    nq = query_y.shape[0]
    nq_pad = ((nq + QB - 1) // QB) * QB
    pad_q = nq_pad - nq
    qidx = (query_y * WIDTH + query_x).astype(jnp.int32)
    qidx_pad = jnp.concatenate(
        [qidx, jnp.arange(pad_q, dtype=jnp.int32)])
    qt_pad = jnp.concatenate(
        [query_t.astype(jnp.float32), jnp.zeros((pad_q,), jnp.float32)])

    G = jnp.pad(jnp.concatenate([CV2[qidx_pad], Cc2[qidx_pad]], axis=1),
                ((0, 0), (0, CC - C2 - 1)))

    O = _recenter(qt_pad, G, T_row, nq_pad)
    out = lax.complex(O[:nq, :D], O[:nq, D:])
    return out
